# Initial kernel scaffold; baseline (speedup 1.0000x reference)
import jax
import jax.numpy as jnp
from jax.experimental import pallas as pl


def kernel(x, edge_index, W1, b1, W2, b2):
    raise NotImplementedError("write your pallas kernel here")


# speedup vs baseline: 7.4255x; 7.4255x over previous
import functools

import jax
import jax.numpy as jnp
from jax import lax
from jax.experimental import pallas as pl
from jax.experimental.pallas import tpu as pltpu
from jax.experimental.pallas import tpu_sc as plsc

N = 10000
NP = 10240
E = 160000
EP = 163840
D = 256
H = 128
NT = 16
RPT = NP // NT
KA = 64
CH_A = (EP // NT) // KA
CH_D = (EP // 32) // 128

_mesh = plsc.VectorSubcoreMesh(core_axis_name="c", subcore_axis_name="s")


@functools.partial(
    pl.kernel,
    out_type=jax.ShapeDtypeStruct((2, NP), jnp.float32),
    mesh=_mesh,
    scratch_types=[
        pltpu.VMEM((CH_D, 128), jnp.int32),
        pltpu.VMEM((128,), jnp.float32),
        pltpu.VMEM((RPT,), jnp.float32),
        pltpu.VMEM_SHARED((NP,), jnp.float32),
    ],
)
def _deg_kernel(dst_hbm, out_hbm, idx_v, ones_v, buf_v, acc):
    c = lax.axis_index("c")
    s = lax.axis_index("s")
    w = s * 2 + c
    init = jnp.where(c == 0, 1.0, 0.0)

    def fill(i, _):
        buf_v[pl.ds(i * 16, 16)] = jnp.broadcast_to(init, (16,))
        return 0

    lax.fori_loop(0, RPT // 16, fill, 0)
    for i in range(8):
        ones_v[pl.ds(i * 16, 16)] = jnp.ones((16,), jnp.float32)
    pltpu.sync_copy(buf_v, acc.at[pl.ds(s * RPT, RPT)])
    pltpu.sync_copy(dst_hbm.at[w], idx_v)
    plsc.subcore_barrier()

    def body(j, _):
        pltpu.sync_copy(ones_v, acc.at[idx_v.at[j]], add=True)
        return 0

    lax.fori_loop(0, CH_D, body, 0)
    plsc.subcore_barrier()
    pltpu.sync_copy(acc.at[pl.ds(s * RPT, RPT)], out_hbm.at[c, pl.ds(s * RPT, RPT)])


@functools.partial(
    pl.kernel,
    out_type=[
        jax.ShapeDtypeStruct((NP, H), jnp.float32),
        jax.ShapeDtypeStruct((NP, H), jnp.float32),
    ],
    mesh=_mesh,
    scratch_types=[
        pltpu.VMEM((8, KA), jnp.int32),
        pltpu.VMEM((8, KA), jnp.int32),
        pltpu.VMEM((KA, H), jnp.float32),
        pltpu.VMEM((KA, H), jnp.float32),
        pltpu.VMEM_SHARED((NP, H), jnp.float32),
        pltpu.SemaphoreType.DMA,
        pltpu.SemaphoreType.DMA,
    ],
)
def _agg_kernel(g0_hbm, g1_hbm, src_hbm, dst_hbm, o0_hbm, o1_hbm,
                sidx, didx, bufa, bufb, acc, sema, semb):
    c = lax.axis_index("c")
    s = lax.axis_index("s")

    def run(g_hbm, o_hbm):
        pltpu.sync_copy(g_hbm.at[pl.ds(s * RPT, RPT)], acc.at[pl.ds(s * RPT, RPT)])
        plsc.subcore_barrier()

        def group(gi, _):
            pltpu.sync_copy(src_hbm.at[s, pl.ds(gi * 8, 8)], sidx)
            pltpu.sync_copy(dst_hbm.at[s, pl.ds(gi * 8, 8)], didx)

            def pair(p, _):
                j = 2 * p
                ca = pltpu.async_copy(g_hbm.at[sidx.at[j]], bufa, sema)
                cb = pltpu.async_copy(g_hbm.at[sidx.at[j + 1]], bufb, semb)
                ca.wait()
                pltpu.sync_copy(bufa, acc.at[didx.at[j]], add=True)
                cb.wait()
                pltpu.sync_copy(bufb, acc.at[didx.at[j + 1]], add=True)
                return 0

            lax.fori_loop(0, 4, pair, 0)
            return 0

        lax.fori_loop(0, CH_A // 8, group, 0)
        plsc.subcore_barrier()
        pltpu.sync_copy(acc.at[pl.ds(s * RPT, RPT)], o_hbm.at[pl.ds(s * RPT, RPT)])

    @pl.when(c == 0)
    def _():
        run(g0_hbm, o0_hbm)

    @pl.when(c == 1)
    def _():
        run(g1_hbm, o1_hbm)


_MB = 512
_GRID = NP // _MB


def _mm1_body(x_ref, w_ref, degp_ref, g0_ref, g1_ref, dinv_ref):
    deg = degp_ref[0, :] + degp_ref[1, :]
    di = lax.rsqrt(deg)
    h = jnp.dot(x_ref[...], w_ref[...], preferred_element_type=jnp.float32)
    g = h * di[:, None]
    g0_ref[...] = g[:, :H]
    g1_ref[...] = g[:, H:]
    dinv_ref[...] = di[None, :]


def _mid_body(a0_ref, a1_ref, dinv_ref, b_ref, w_ref, g0_ref, g1_ref):
    di = dinv_ref[0, :]
    agg = jnp.concatenate([a0_ref[...], a1_ref[...]], axis=1)
    x2 = jnp.maximum(agg * di[:, None] + b_ref[...], 0.0)
    h = jnp.dot(x2, w_ref[...], preferred_element_type=jnp.float32)
    g = h * di[:, None]
    g0_ref[...] = g[:, :H]
    g1_ref[...] = g[:, H:]


def _fin_body(a0_ref, a1_ref, dinv_ref, b_ref, o_ref):
    di = dinv_ref[0, :]
    agg = jnp.concatenate([a0_ref[...], a1_ref[...]], axis=1)
    o_ref[...] = agg * di[:, None] + b_ref[...]


def _mm1(x_p, w1, degp):
    return pl.pallas_call(
        _mm1_body,
        grid=(_GRID,),
        in_specs=[
            pl.BlockSpec((_MB, D), lambda i: (i, 0)),
            pl.BlockSpec((D, D), lambda i: (0, 0)),
            pl.BlockSpec((2, _MB), lambda i: (0, i)),
        ],
        out_specs=[
            pl.BlockSpec((_MB, H), lambda i: (i, 0)),
            pl.BlockSpec((_MB, H), lambda i: (i, 0)),
            pl.BlockSpec((1, _MB), lambda i: (0, i)),
        ],
        out_shape=[
            jax.ShapeDtypeStruct((NP, H), jnp.float32),
            jax.ShapeDtypeStruct((NP, H), jnp.float32),
            jax.ShapeDtypeStruct((1, NP), jnp.float32),
        ],
    )(x_p, w1, degp)


def _mid(a0, a1, dinv, b1, w2):
    return pl.pallas_call(
        _mid_body,
        grid=(_GRID,),
        in_specs=[
            pl.BlockSpec((_MB, H), lambda i: (i, 0)),
            pl.BlockSpec((_MB, H), lambda i: (i, 0)),
            pl.BlockSpec((1, _MB), lambda i: (0, i)),
            pl.BlockSpec((1, D), lambda i: (0, 0)),
            pl.BlockSpec((D, D), lambda i: (0, 0)),
        ],
        out_specs=[
            pl.BlockSpec((_MB, H), lambda i: (i, 0)),
            pl.BlockSpec((_MB, H), lambda i: (i, 0)),
        ],
        out_shape=[
            jax.ShapeDtypeStruct((NP, H), jnp.float32),
            jax.ShapeDtypeStruct((NP, H), jnp.float32),
        ],
    )(a0, a1, dinv, b1, w2)


def _fin(a0, a1, dinv, b2):
    return pl.pallas_call(
        _fin_body,
        grid=(_GRID,),
        in_specs=[
            pl.BlockSpec((_MB, H), lambda i: (i, 0)),
            pl.BlockSpec((_MB, H), lambda i: (i, 0)),
            pl.BlockSpec((1, _MB), lambda i: (0, i)),
            pl.BlockSpec((1, D), lambda i: (0, 0)),
        ],
        out_specs=pl.BlockSpec((_MB, D), lambda i: (i, 0)),
        out_shape=jax.ShapeDtypeStruct((NP, D), jnp.float32),
    )(a0, a1, dinv, b2)


@jax.jit
def kernel(x, edge_index, W1, b1, W2, b2):
    src = edge_index[0].astype(jnp.int32)
    dst = edge_index[1].astype(jnp.int32)
    pad = EP - E
    src_p = jnp.concatenate([src, jnp.zeros((pad,), jnp.int32)])
    dst_p = jnp.concatenate([dst, jnp.full((pad,), N, jnp.int32)])
    src3 = src_p.reshape(NT, CH_A, KA)
    dst3 = dst_p.reshape(NT, CH_A, KA)
    dst3d = dst_p.reshape(32, CH_D, 128)
    x_p = jnp.pad(x, ((0, NP - N), (0, 0)))

    degp = _deg_kernel(dst3d)
    g0, g1, dinv = _mm1(x_p, W1, degp)
    a0, a1 = _agg_kernel(g0, g1, src3, dst3)
    g0, g1 = _mid(a0, a1, dinv, b1[None, :], W2)
    a0, a1 = _agg_kernel(g0, g1, src3, dst3)
    out = _fin(a0, a1, dinv, b2[None, :])
    return out[:N]
